# baseline (speedup 1.0000x reference)
import functools

import jax
import jax.numpy as jnp
from jax import lax
from jax.experimental import pallas as pl
from jax.experimental.pallas import tpu as pltpu
from jax.experimental.pallas import tpu_sc as plsc

_NC = 2
_NS = 16
_NW = _NC * _NS
_C = 128
_G = 4
_R = 1
_NBUF = _G


@functools.cache
def _emb_lookup(V, D, NCH):
    vmesh = plsc.VectorSubcoreMesh(core_axis_name="c", subcore_axis_name="s")
    smesh = plsc.ScalarSubcoreMesh(axis_name="c")
    ngroups = NCH // _G
    nrounds = ngroups * _R

    VMEM = pltpu.MemorySpace.VMEM
    SEM = pltpu.MemorySpace.SEMAPHORE
    dma_t = pltpu.SemaphoreType.DMA.dtype
    reg_t = pltpu.SemaphoreType.REGULAR.dtype

    scratch = [
        (VMEM @ vmesh)((NCH, _C), jnp.int32),
        (VMEM @ vmesh)((_NBUF, _C, D), jnp.float32),
        pltpu.VMEM_SHARED((_NS, _C, D), jnp.float32),
        (SEM @ vmesh)((_NBUF,), dma_t),
        (SEM @ vmesh)((_NBUF,), dma_t),
        (SEM @ vmesh)((), reg_t),
        (SEM @ smesh)((), reg_t),
        (SEM @ smesh)((), dma_t),
    ]

    def tec(idx_hbm, table_hbm, out_hbm, idx_v, rows_v, slots, gsem, osem,
            free_sem, full_sem, dsem):
        cid = lax.axis_index("c")
        sid = lax.axis_index("s")
        pltpu.sync_copy(idx_hbm.at[sid, cid], idx_v)

        for b in range(_NBUF):
            pltpu.async_copy(table_hbm.at[idx_v.at[b]], rows_v.at[b], gsem.at[b])

        def group(g, carry):
            base = g * _G
            for b in range(_G):
                j = base + b
                pltpu.make_async_copy(
                    table_hbm.at[idx_v.at[j]], rows_v.at[b], gsem.at[b]
                ).wait()
                if b < _G - _R:
                    pltpu.async_copy(
                        rows_v.at[b], out_hbm.at[sid, cid, j], osem.at[b]
                    )
                else:
                    @pl.when((g > 0) & (sid == 0))
                    def _():
                        pl.semaphore_wait(free_sem, 1)
                    plsc.subcore_barrier()
                    pltpu.async_copy(rows_v.at[b], slots.at[sid], osem.at[b])

            @pl.when(g < ngroups - 1)
            def _():
                for b in range(_G):
                    pltpu.make_async_copy(
                        rows_v.at[b],
                        out_hbm.at[sid, cid, base + b]
                        if b < _G - _R
                        else slots.at[sid],
                        osem.at[b],
                    ).wait()
                    if b >= _G - _R:
                        pl.semaphore_signal(full_sem, 1)
                    pltpu.async_copy(
                        table_hbm.at[idx_v.at[base + _G + b]],
                        rows_v.at[b],
                        gsem.at[b],
                    )

            return carry

        lax.fori_loop(0, ngroups, group, 0)

        last = (ngroups - 1) * _G
        for b in range(_G):
            pltpu.make_async_copy(
                rows_v.at[b],
                out_hbm.at[sid, cid, last + b]
                if b < _G - _R
                else slots.at[sid],
                osem.at[b],
            ).wait()
            if b >= _G - _R:
                pl.semaphore_signal(full_sem, 1)
        @pl.when(sid == 0)
        def _():
            pl.semaphore_wait(free_sem, 1)

    def scs(idx_hbm, table_hbm, out_hbm, idx_v, rows_v, slots, gsem, osem,
            free_sem, full_sem, dsem):
        cid = lax.axis_index("c")

        def rnd(r, carry):
            g = r // _R
            j = g * _G + (_G - _R) + lax.rem(r, _R)
            pl.semaphore_wait(full_sem, _NS)
            pltpu.async_copy(slots, out_hbm.at[:, cid, j], dsem)
            pltpu.make_async_copy(slots, out_hbm.at[:, cid, j], dsem).wait()
            pl.semaphore_signal(free_sem, 1, device_id={"s": 0})
            return carry

        lax.fori_loop(0, nrounds, rnd, 0)

    return pl.kernel(
        body=[tec, scs],
        mesh=[vmesh, smesh],
        out_type=jax.ShapeDtypeStruct((_NS, _NC, NCH, _C, D), jnp.float32),
        scratch_types=scratch,
    )


def kernel(x, embedding):
    V, D = embedding.shape
    idx = x.reshape(_NS, _NC, -1, _C).astype(jnp.int32)
    NCH = idx.shape[2]
    out = _emb_lookup(V, D, NCH)(idx, embedding)
    return out.reshape(*x.shape, D)


# speedup vs baseline: 1.0165x; 1.0165x over previous
import functools

import jax
import jax.numpy as jnp
from jax import lax
from jax.experimental import pallas as pl
from jax.experimental.pallas import tpu as pltpu
from jax.experimental.pallas import tpu_sc as plsc

_NC = 2
_NS = 16
_NW = _NC * _NS
_C = 128
_G = 4
_R = 1
_NBUF = _G


@functools.cache
def _emb_lookup(V, D, NCH):
    vmesh = plsc.VectorSubcoreMesh(core_axis_name="c", subcore_axis_name="s")
    smesh = plsc.ScalarSubcoreMesh(axis_name="c")
    ngroups = NCH // _G
    nrounds = ngroups * _R

    VMEM = pltpu.MemorySpace.VMEM
    SEM = pltpu.MemorySpace.SEMAPHORE
    dma_t = pltpu.SemaphoreType.DMA.dtype
    reg_t = pltpu.SemaphoreType.REGULAR.dtype

    scratch = [
        (VMEM @ vmesh)((NCH, _C), jnp.int32),
        (VMEM @ vmesh)((_NBUF, _C, D), jnp.float32),
        pltpu.VMEM_SHARED((_NS, _C, D), jnp.float32),
        (SEM @ vmesh)((_NBUF,), dma_t),
        (SEM @ vmesh)((_NBUF,), dma_t),
        (SEM @ vmesh)((), reg_t),
        (SEM @ smesh)((), reg_t),
        (SEM @ smesh)((), dma_t),
    ]

    def tec(idx_hbm, table_hbm, out_hbm, idx_v, rows_v, slots, gsem, osem,
            free_sem, full_sem, dsem):
        cid = lax.axis_index("c")
        sid = lax.axis_index("s")
        pltpu.sync_copy(idx_hbm.at[sid, cid], idx_v)

        for b in range(_NBUF):
            pltpu.async_copy(table_hbm.at[idx_v.at[b]], rows_v.at[b], gsem.at[b])

        def group(g, carry):
            base = g * _G
            @pl.when((g > 0) & (sid == 0))
            def _():
                pl.semaphore_wait(free_sem, 1)
            plsc.subcore_barrier()
            pltpu.make_async_copy(
                table_hbm.at[idx_v.at[base]], rows_v.at[0], gsem.at[0]
            ).wait()
            pltpu.async_copy(rows_v.at[0], slots.at[sid], osem.at[0])
            for b in range(1, _G):
                j = base + b
                pltpu.make_async_copy(
                    table_hbm.at[idx_v.at[j]], rows_v.at[b], gsem.at[b]
                ).wait()
                pltpu.async_copy(
                    rows_v.at[b], out_hbm.at[sid, cid, j], osem.at[b]
                )
                if b == 1:
                    pltpu.make_async_copy(
                        rows_v.at[0], slots.at[sid], osem.at[0]
                    ).wait()
                    pl.semaphore_signal(full_sem, 1)

            @pl.when(g < ngroups - 1)
            def _():
                for b in range(_G):
                    if b > 0:
                        pltpu.make_async_copy(
                            rows_v.at[b], out_hbm.at[sid, cid, base + b],
                            osem.at[b],
                        ).wait()
                    pltpu.async_copy(
                        table_hbm.at[idx_v.at[base + _G + b]],
                        rows_v.at[b],
                        gsem.at[b],
                    )

            return carry

        lax.fori_loop(0, ngroups, group, 0)

        last = (ngroups - 1) * _G
        for b in range(1, _G):
            pltpu.make_async_copy(
                rows_v.at[b], out_hbm.at[sid, cid, last + b], osem.at[b]
            ).wait()
        @pl.when(sid == 0)
        def _():
            pl.semaphore_wait(free_sem, 1)

    def scs(idx_hbm, table_hbm, out_hbm, idx_v, rows_v, slots, gsem, osem,
            free_sem, full_sem, dsem):
        cid = lax.axis_index("c")

        def rnd(r, carry):
            g = r // _R
            j = g * _G
            pl.semaphore_wait(full_sem, _NS)
            pltpu.async_copy(slots, out_hbm.at[:, cid, j], dsem)
            pltpu.make_async_copy(slots, out_hbm.at[:, cid, j], dsem).wait()
            pl.semaphore_signal(free_sem, 1, device_id={"s": 0})
            return carry

        lax.fori_loop(0, nrounds, rnd, 0)

    return pl.kernel(
        body=[tec, scs],
        mesh=[vmesh, smesh],
        out_type=jax.ShapeDtypeStruct((_NS, _NC, NCH, _C, D), jnp.float32),
        scratch_types=scratch,
    )


def kernel(x, embedding):
    V, D = embedding.shape
    idx = x.reshape(_NS, _NC, -1, _C).astype(jnp.int32)
    NCH = idx.shape[2]
    out = _emb_lookup(V, D, NCH)(idx, embedding)
    return out.reshape(*x.shape, D)


# speedup vs baseline: 1.0214x; 1.0048x over previous
import functools

import jax
import jax.numpy as jnp
from jax import lax
from jax.experimental import pallas as pl
from jax.experimental.pallas import tpu as pltpu
from jax.experimental.pallas import tpu_sc as plsc

_NC = 2
_NS = 16
_NW = _NC * _NS
_C = 128
_G = 4
_R = 1
_NBUF = _G


@functools.cache
def _emb_lookup(V, D, NCH):
    vmesh = plsc.VectorSubcoreMesh(core_axis_name="c", subcore_axis_name="s")
    smesh = plsc.ScalarSubcoreMesh(axis_name="c")
    ngroups = NCH // _G
    nrounds = ngroups * _R

    VMEM = pltpu.MemorySpace.VMEM
    SEM = pltpu.MemorySpace.SEMAPHORE
    dma_t = pltpu.SemaphoreType.DMA.dtype
    reg_t = pltpu.SemaphoreType.REGULAR.dtype

    scratch = [
        (VMEM @ vmesh)((NCH, _C), jnp.int32),
        (VMEM @ vmesh)((_NBUF, _C, D), jnp.float32),
        pltpu.VMEM_SHARED((_NS, _C, D), jnp.float32),
        (SEM @ vmesh)((_NBUF,), dma_t),
        (SEM @ vmesh)((_NBUF,), dma_t),
        (SEM @ vmesh)((), reg_t),
        (SEM @ smesh)((), reg_t),
        (SEM @ smesh)((), dma_t),
    ]

    def tec(idx_hbm, table_hbm, out_hbm, idx_v, rows_v, slots, gsem, osem,
            free_sem, full_sem, dsem):
        cid = lax.axis_index("c")
        sid = lax.axis_index("s")
        pltpu.sync_copy(idx_hbm.at[sid, cid], idx_v)

        for b in range(_NBUF):
            pltpu.async_copy(table_hbm.at[idx_v.at[b]], rows_v.at[b], gsem.at[b])

        def group(g, carry):
            base = g * _G
            @pl.when((g > 0) & (sid == 0))
            def _():
                pl.semaphore_wait(free_sem, 1)
            plsc.subcore_barrier()
            pltpu.make_async_copy(
                table_hbm.at[idx_v.at[base]], rows_v.at[0], gsem.at[0]
            ).wait()
            pltpu.async_copy(rows_v.at[0], slots.at[sid], osem.at[0])
            for b in range(1, _G):
                j = base + b
                pltpu.make_async_copy(
                    table_hbm.at[idx_v.at[j]], rows_v.at[b], gsem.at[b]
                ).wait()
                pltpu.async_copy(
                    rows_v.at[b], out_hbm.at[sid, cid, j], osem.at[b]
                )
                if b == 1:
                    pltpu.make_async_copy(
                        rows_v.at[0], slots.at[sid], osem.at[0]
                    ).wait()
                    pl.semaphore_signal(full_sem, 1)
                    @pl.when(g < ngroups - 1)
                    def _():
                        pltpu.async_copy(
                            table_hbm.at[idx_v.at[base + _G]],
                            rows_v.at[0],
                            gsem.at[0],
                        )

            @pl.when(g < ngroups - 1)
            def _():
                for b in range(1, _G):
                    pltpu.make_async_copy(
                        rows_v.at[b], out_hbm.at[sid, cid, base + b],
                        osem.at[b],
                    ).wait()
                    pltpu.async_copy(
                        table_hbm.at[idx_v.at[base + _G + b]],
                        rows_v.at[b],
                        gsem.at[b],
                    )

            return carry

        lax.fori_loop(0, ngroups, group, 0)

        last = (ngroups - 1) * _G
        for b in range(1, _G):
            pltpu.make_async_copy(
                rows_v.at[b], out_hbm.at[sid, cid, last + b], osem.at[b]
            ).wait()
        @pl.when(sid == 0)
        def _():
            pl.semaphore_wait(free_sem, 1)

    def scs(idx_hbm, table_hbm, out_hbm, idx_v, rows_v, slots, gsem, osem,
            free_sem, full_sem, dsem):
        cid = lax.axis_index("c")

        def rnd(r, carry):
            g = r // _R
            j = g * _G
            pl.semaphore_wait(full_sem, _NS)
            pltpu.async_copy(slots, out_hbm.at[:, cid, j], dsem)
            pltpu.make_async_copy(slots, out_hbm.at[:, cid, j], dsem).wait()
            pl.semaphore_signal(free_sem, 1, device_id={"s": 0})
            return carry

        lax.fori_loop(0, nrounds, rnd, 0)

    return pl.kernel(
        body=[tec, scs],
        mesh=[vmesh, smesh],
        out_type=jax.ShapeDtypeStruct((_NS, _NC, NCH, _C, D), jnp.float32),
        scratch_types=scratch,
    )


def kernel(x, embedding):
    V, D = embedding.shape
    idx = x.reshape(_NS, _NC, -1, _C).astype(jnp.int32)
    NCH = idx.shape[2]
    out = _emb_lookup(V, D, NCH)(idx, embedding)
    return out.reshape(*x.shape, D)
